# Initial kernel scaffold; baseline (speedup 1.0000x reference)
import jax
import jax.numpy as jnp
from jax.experimental import pallas as pl


def kernel(x, edge_index, W1, b1, W2, b2, gamma, beta, Wlin, blin):
    raise NotImplementedError("write your pallas kernel here")


# speedup vs baseline: 7.2296x; 7.2296x over previous
import functools

import jax
import jax.numpy as jnp
from jax import lax
from jax.experimental import pallas as pl
from jax.experimental.pallas import tpu as pltpu
from jax.experimental.pallas import tpu_sc as plsc

_NC = 2
_NS = 16
_NW = _NC * _NS
_L = 16
_BN_EPS = 1e-5


def _chunk_size(n):
    for c in range(128, 7, -8):
        if n % c == 0:
            return c
    raise ValueError(f"no aligned chunk size for {n}")


def _row_chunk(rows):
    for c in range(128, 7, -8):
        if rows % c == 0:
            return c
    raise ValueError(f"no aligned row chunk for {rows}")


@functools.lru_cache(maxsize=None)
def _hist_call(N, E, HW):
    EPW = E // _NW
    C = _chunk_size(EPW)
    NCH = EPW // C
    ZR = _row_chunk(N)
    NZ = N // ZR
    NZI = (NZ + _NS - 1) // _NS
    mesh = plsc.VectorSubcoreMesh(core_axis_name="c", subcore_axis_name="s", num_cores=_NC, num_subcores=_NS)

    @functools.partial(
        pl.kernel,
        mesh=mesh,
        out_type=jax.ShapeDtypeStruct((_NC, N, HW), jnp.float32),
        scratch_types=[
            pltpu.VMEM_SHARED((N, HW), jnp.float32),
            pltpu.VMEM((ZR, HW), jnp.float32),
            pltpu.VMEM((C,), jnp.int32),
            pltpu.VMEM((C, HW), jnp.float32),
        ],
    )
    def hist(src_hbm, out_hbm, acc_sp, z_v, idx_v, ones_v):
        cid = lax.axis_index("c")
        sid = lax.axis_index("s")
        wid = sid * _NC + cid

        def fill_z(i, _):
            for k in range(HW // _L):
                z_v[i, pl.ds(k * _L, _L)] = jnp.zeros((_L,), jnp.float32)
            return 0

        lax.fori_loop(0, ZR, fill_z, 0)

        def fill_o(i, _):
            for k in range(HW // _L):
                ones_v[i, pl.ds(k * _L, _L)] = jnp.ones((_L,), jnp.float32)
            return 0

        lax.fori_loop(0, C, fill_o, 0)

        def zero_acc(i, _):
            k = i * _NS + sid

            @pl.when(k < NZ)
            def _():
                pltpu.sync_copy(z_v, acc_sp.at[pl.ds(k * ZR, ZR)])

            return 0

        lax.fori_loop(0, NZI, zero_acc, 0)
        plsc.subcore_barrier()

        base = wid * EPW

        def step(j, _):
            pltpu.sync_copy(src_hbm.at[pl.ds(base + j * C, C)], idx_v)
            pltpu.sync_copy(ones_v, acc_sp.at[idx_v], add=True)
            return 0

        lax.fori_loop(0, NCH, step, 0)
        plsc.subcore_barrier()

        def readout(i, _):
            k = i * _NS + sid

            @pl.when(k < NZ)
            def _():
                pltpu.sync_copy(acc_sp.at[pl.ds(k * ZR, ZR)], z_v)
                pltpu.sync_copy(z_v, out_hbm.at[cid, pl.ds(k * ZR, ZR)])

            return 0

        lax.fori_loop(0, NZI, readout, 0)

    return hist


@functools.lru_cache(maxsize=None)
def _scatter_call(N, E, F):
    EPW = E // _NW
    C = _chunk_size(EPW)
    NCH = EPW // C
    ZR = _row_chunk(N)
    NZ = N // ZR
    NZI = (NZ + _NS - 1) // _NS
    mesh = plsc.VectorSubcoreMesh(core_axis_name="c", subcore_axis_name="s", num_cores=_NC, num_subcores=_NS)

    @functools.partial(
        pl.kernel,
        mesh=mesh,
        out_type=jax.ShapeDtypeStruct((_NC, N, F), jnp.float32),
        scratch_types=[
            pltpu.VMEM_SHARED((N, F), jnp.float32),
            pltpu.VMEM((ZR, F), jnp.float32),
            pltpu.VMEM((C,), jnp.int32),
            pltpu.VMEM((C,), jnp.int32),
            pltpu.VMEM((C, F), jnp.float32),
            pltpu.SemaphoreType.DMA,
        ],
    )
    def scat(u_hbm, src_hbm, dst_hbm, out_hbm, acc_sp, z_v, src_v, dst_v,
             rows_v, sem):
        cid = lax.axis_index("c")
        sid = lax.axis_index("s")
        wid = sid * _NC + cid

        def fill_z(i, _):
            for k in range(F // _L):
                z_v[i, pl.ds(k * _L, _L)] = jnp.zeros((_L,), jnp.float32)
            return 0

        lax.fori_loop(0, ZR, fill_z, 0)

        def zero_acc(i, _):
            k = i * _NS + sid

            @pl.when(k < NZ)
            def _():
                pltpu.sync_copy(z_v, acc_sp.at[pl.ds(k * ZR, ZR)])

            return 0

        lax.fori_loop(0, NZI, zero_acc, 0)
        plsc.subcore_barrier()

        base = wid * EPW

        def step(j, _):
            off = base + j * C
            pltpu.sync_copy(src_hbm.at[pl.ds(off, C)], src_v)
            pltpu.sync_copy(dst_hbm.at[pl.ds(off, C)], dst_v)
            pltpu.async_copy(u_hbm.at[src_v], rows_v, sem).wait()
            pltpu.sync_copy(rows_v, acc_sp.at[dst_v], add=True)
            return 0

        lax.fori_loop(0, NCH, step, 0)
        plsc.subcore_barrier()

        def readout(i, _):
            k = i * _NS + sid

            @pl.when(k < NZ)
            def _():
                pltpu.sync_copy(acc_sp.at[pl.ds(k * ZR, ZR)], z_v)
                pltpu.sync_copy(z_v, out_hbm.at[cid, pl.ds(k * ZR, ZR)])

            return 0

        lax.fori_loop(0, NZI, readout, 0)

    return scat


def _prep_body(hist_ref, x_ref, dinv_ref, u0_ref):
    deg = hist_ref[0, :, :_L] + hist_ref[1, :, :_L]
    dinv = jnp.where(deg > 0, 1.0 / jnp.sqrt(jnp.maximum(deg, 1e-12)), 0.0)
    dinv_ref[...] = dinv
    u0_ref[...] = x_ref[...] * dinv[:, :1]


def _mid_body(p_ref, dinv_ref, x_ref, w_ref, u1_ref, acc_ref):
    d = dinv_ref[:, :1]
    tx1 = -d * (p_ref[0] + p_ref[1])
    u1_ref[...] = d * tx1
    acc_ref[...] = (
        jnp.dot(x_ref[...], w_ref[0], preferred_element_type=jnp.float32)
        + jnp.dot(tx1, w_ref[1], preferred_element_type=jnp.float32)
    )


def _bn_body(q_ref, dinv_ref, x_ref, acc_ref, w_ref, b_ref, g_ref, be_ref,
             hbn_ref, u0_ref):
    d = dinv_ref[:, :1]
    tx2 = -2.0 * d * (q_ref[0] + q_ref[1]) - x_ref[...]
    h = acc_ref[...] + jnp.dot(tx2, w_ref[2], preferred_element_type=jnp.float32)
    h = jnp.maximum(h + b_ref[...], 0.0)
    mean = jnp.mean(h, axis=0, keepdims=True)
    var = jnp.mean((h - mean) ** 2, axis=0, keepdims=True)
    hbn = (h - mean) / jnp.sqrt(var + _BN_EPS) * g_ref[...] + be_ref[...]
    hbn_ref[...] = hbn
    u0_ref[...] = d * hbn


def _final_body(t_ref, dinv_ref, x_ref, acc_ref, w_ref, b_ref, wl_ref, bl_ref,
                out_ref):
    d = dinv_ref[:, :1]
    tx2 = -2.0 * d * (t_ref[0] + t_ref[1]) - x_ref[...]
    h = acc_ref[...] + jnp.dot(tx2, w_ref[2], preferred_element_type=jnp.float32)
    h = jnp.maximum(h + b_ref[...], 0.0)
    out_ref[...] = (
        lax.dot_general(h, wl_ref[...], (((1,), (1,)), ((), ())),
                        preferred_element_type=jnp.float32)
        + bl_ref[...]
    )


def _tc(body, out_shapes):
    return pl.pallas_call(body, out_shape=out_shapes)


def kernel(x, edge_index, W1, b1, W2, b2, gamma, beta, Wlin, blin):
    N, F = x.shape
    E = edge_index.shape[1]
    OUT_F = Wlin.shape[0]
    f32 = jnp.float32

    b1r = b1.reshape(1, -1)
    b2r = b2.reshape(1, -1)
    gr = gamma.reshape(1, -1)
    ber = beta.reshape(1, -1)
    blr = blin.reshape(1, -1)

    src = edge_index[0]
    dst = edge_index[1]

    hist = _hist_call(N, E, F)(src)
    dinv, u0 = _tc(_prep_body, (
        jax.ShapeDtypeStruct((N, _L), f32),
        jax.ShapeDtypeStruct((N, F), f32),
    ))(hist, x)

    scatter = _scatter_call(N, E, F)

    p = scatter(u0, src, dst)
    u1, acc1 = _tc(_mid_body, (
        jax.ShapeDtypeStruct((N, F), f32),
        jax.ShapeDtypeStruct((N, F), f32),
    ))(p, dinv, x, W1)
    q = scatter(u1, src, dst)
    hbn, u0b = _tc(_bn_body, (
        jax.ShapeDtypeStruct((N, F), f32),
        jax.ShapeDtypeStruct((N, F), f32),
    ))(q, dinv, x, acc1, W1, b1r, gr, ber)

    r = scatter(u0b, src, dst)
    u1b, acc2 = _tc(_mid_body, (
        jax.ShapeDtypeStruct((N, F), f32),
        jax.ShapeDtypeStruct((N, F), f32),
    ))(r, dinv, hbn, W2)
    t = scatter(u1b, src, dst)
    out = _tc(_final_body, jax.ShapeDtypeStruct((N, OUT_F), f32))(
        t, dinv, hbn, acc2, W2, b2r, Wlin, blr)
    return out


# speedup vs baseline: 14.9987x; 2.0746x over previous
import functools

import jax
import jax.numpy as jnp
from jax import lax
from jax.experimental import pallas as pl
from jax.experimental.pallas import tpu as pltpu
from jax.experimental.pallas import tpu_sc as plsc

_NC = 2
_NS = 16
_NW = _NC * _NS
_L = 16
_BN_EPS = 1e-5


def _chunk_size(n, nb=1, cap=64):
    for c in range(cap, 7, -8):
        if n % c == 0 and (n // c) % nb == 0:
            return c
    raise ValueError(f"no aligned chunk size for {n}")


def _row_chunk(rows):
    for c in range(64, 7, -8):
        if rows % c == 0:
            return c
    raise ValueError(f"no aligned row chunk for {rows}")


@functools.lru_cache(maxsize=None)
def _hist_call(N, E, HW):
    EPW = E // _NW
    C = _chunk_size(EPW)
    NCH = EPW // C
    ZR = _row_chunk(N)
    NZ = N // ZR
    NZI = (NZ + _NS - 1) // _NS
    mesh = plsc.VectorSubcoreMesh(core_axis_name="c", subcore_axis_name="s", num_cores=_NC, num_subcores=_NS)

    @functools.partial(
        pl.kernel,
        mesh=mesh,
        out_type=jax.ShapeDtypeStruct((_NC, N, HW), jnp.float32),
        scratch_types=[
            pltpu.VMEM_SHARED((N, HW), jnp.float32),
            pltpu.VMEM((ZR, HW), jnp.float32),
            pltpu.VMEM((C,), jnp.int32),
            pltpu.VMEM((C, HW), jnp.float32),
        ],
    )
    def hist(src_hbm, out_hbm, acc_sp, z_v, idx_v, ones_v):
        cid = lax.axis_index("c")
        sid = lax.axis_index("s")
        wid = sid * _NC + cid

        def fill_z(i, _):
            for k in range(HW // _L):
                z_v[i, pl.ds(k * _L, _L)] = jnp.zeros((_L,), jnp.float32)
            return 0

        lax.fori_loop(0, ZR, fill_z, 0)

        def fill_o(i, _):
            for k in range(HW // _L):
                ones_v[i, pl.ds(k * _L, _L)] = jnp.ones((_L,), jnp.float32)
            return 0

        lax.fori_loop(0, C, fill_o, 0)

        def zero_acc(i, _):
            k = i * _NS + sid

            @pl.when(k < NZ)
            def _():
                pltpu.sync_copy(z_v, acc_sp.at[pl.ds(k * ZR, ZR)])

            return 0

        lax.fori_loop(0, NZI, zero_acc, 0)
        plsc.subcore_barrier()

        base = wid * EPW

        def step(j, _):
            pltpu.sync_copy(src_hbm.at[pl.ds(base + j * C, C)], idx_v)
            pltpu.sync_copy(ones_v, acc_sp.at[idx_v], add=True)
            return 0

        lax.fori_loop(0, NCH, step, 0)
        plsc.subcore_barrier()

        def readout(i, _):
            k = i * _NS + sid

            @pl.when(k < NZ)
            def _():
                pltpu.sync_copy(acc_sp.at[pl.ds(k * ZR, ZR)], z_v)
                pltpu.sync_copy(z_v, out_hbm.at[cid, pl.ds(k * ZR, ZR)])

            return 0

        lax.fori_loop(0, NZI, readout, 0)

    return hist


@functools.lru_cache(maxsize=None)
def _scatter_call(N, E, F):
    EPW = E // _NW
    NB = 5
    C = _chunk_size(EPW, nb=NB)
    NCH = EPW // C
    ZR = _row_chunk(N)
    NZ = N // ZR
    NZI = (NZ + _NS - 1) // _NS
    mesh = plsc.VectorSubcoreMesh(core_axis_name="c", subcore_axis_name="s", num_cores=_NC, num_subcores=_NS)

    @functools.partial(
        pl.kernel,
        mesh=mesh,
        out_type=jax.ShapeDtypeStruct((_NC, N, F), jnp.float32),
        scratch_types=(
            [pltpu.VMEM_SHARED((N, F), jnp.float32),
             pltpu.VMEM((ZR, F), jnp.float32)]
            + [pltpu.VMEM((C,), jnp.int32)] * NB
            + [pltpu.VMEM((C,), jnp.int32)] * NB
            + [pltpu.VMEM((C, F), jnp.float32)] * NB
            + [pltpu.SemaphoreType.DMA] * 4
        ),
    )
    def scat(u_hbm, src_hbm, dst_hbm, out_hbm, acc_sp, z_v, *bufs):
        srcb = bufs[:NB]
        dstb = bufs[NB:2 * NB]
        rows = bufs[2 * NB:3 * NB]
        gsem, ssem, xsem, dsem = bufs[3 * NB:]
        cid = lax.axis_index("c")
        sid = lax.axis_index("s")
        wid = sid * _NC + cid
        base = wid * EPW

        def fill_z(i, _):
            for k in range(F // _L):
                z_v[i, pl.ds(k * _L, _L)] = jnp.zeros((_L,), jnp.float32)
            return 0

        lax.fori_loop(0, ZR, fill_z, 0)

        def zero_acc(i, _):
            k = i * _NS + sid

            @pl.when(k < NZ)
            def _():
                pltpu.sync_copy(z_v, acc_sp.at[pl.ds(k * ZR, ZR)])

            return 0

        lax.fori_loop(0, NZI, zero_acc, 0)
        plsc.subcore_barrier()

        def src_cp(j, b):
            return pltpu.make_async_copy(
                src_hbm.at[pl.ds(base + j * C, C)], srcb[b], xsem)

        def dst_cp(j, b):
            return pltpu.make_async_copy(
                dst_hbm.at[pl.ds(base + j * C, C)], dstb[b], dsem)

        def gat_cp(b):
            return pltpu.make_async_copy(u_hbm.at[srcb[b]], rows[b], gsem)

        def sca_start(b):
            pltpu.async_copy(rows[b], acc_sp.at[dstb[b]], ssem, add=True)

        def sca_wait(b):
            pltpu.make_async_copy(rows[b], acc_sp.at[dstb[b]], ssem).wait()

        for m in range(NB):
            src_cp(m, m).start()
        for m in range(3):
            dst_cp(m, m).start()
        for m in range(2):
            src_cp(m, m).wait()
            gat_cp(m).start()

        def block(k, _):
            for b in range(NB):
                j = k * NB + b

                @pl.when(j >= 2)
                def _(b2=(b - 2) % NB):
                    sca_wait(b2)

                @pl.when(j + 2 < NCH)
                def _(b2=(b + 2) % NB, j2=j + 2):
                    src_cp(j2, b2).wait()
                    gat_cp(b2).start()

                gat_cp(b).wait()
                dst_cp(j, b).wait()
                sca_start(b)

                @pl.when(j + NB < NCH)
                def _(j2=j + NB):
                    src_cp(j2, b).start()

                @pl.when(j + 3 < NCH)
                def _(b2=(b + 3) % NB, j2=j + 3):
                    dst_cp(j2, b2).start()

            return 0

        lax.fori_loop(0, NCH // NB, block, 0)

        sca_wait((NCH - 2) % NB)
        sca_wait((NCH - 1) % NB)
        plsc.subcore_barrier()

        def readout(i, _):
            k = i * _NS + sid

            @pl.when(k < NZ)
            def _():
                pltpu.sync_copy(acc_sp.at[pl.ds(k * ZR, ZR)], z_v)
                pltpu.sync_copy(z_v, out_hbm.at[cid, pl.ds(k * ZR, ZR)])

            return 0

        lax.fori_loop(0, NZI, readout, 0)

    return scat


def _prep_body(hist_ref, x_ref, dinv_ref, u0_ref):
    deg = hist_ref[0, :, :_L] + hist_ref[1, :, :_L]
    dinv = jnp.where(deg > 0, 1.0 / jnp.sqrt(jnp.maximum(deg, 1e-12)), 0.0)
    dinv_ref[...] = dinv
    u0_ref[...] = x_ref[...] * dinv[:, :1]


def _mid_body(p_ref, dinv_ref, x_ref, w_ref, u1_ref, acc_ref):
    d = dinv_ref[:, :1]
    tx1 = -d * (p_ref[0] + p_ref[1])
    u1_ref[...] = d * tx1
    acc_ref[...] = (
        jnp.dot(x_ref[...], w_ref[0], preferred_element_type=jnp.float32)
        + jnp.dot(tx1, w_ref[1], preferred_element_type=jnp.float32)
    )


def _bn_body(q_ref, dinv_ref, x_ref, acc_ref, w_ref, b_ref, g_ref, be_ref,
             hbn_ref, u0_ref):
    d = dinv_ref[:, :1]
    tx2 = -2.0 * d * (q_ref[0] + q_ref[1]) - x_ref[...]
    h = acc_ref[...] + jnp.dot(tx2, w_ref[2], preferred_element_type=jnp.float32)
    h = jnp.maximum(h + b_ref[...], 0.0)
    mean = jnp.mean(h, axis=0, keepdims=True)
    var = jnp.mean((h - mean) ** 2, axis=0, keepdims=True)
    hbn = (h - mean) / jnp.sqrt(var + _BN_EPS) * g_ref[...] + be_ref[...]
    hbn_ref[...] = hbn
    u0_ref[...] = d * hbn


def _final_body(t_ref, dinv_ref, x_ref, acc_ref, w_ref, b_ref, wl_ref, bl_ref,
                out_ref):
    d = dinv_ref[:, :1]
    tx2 = -2.0 * d * (t_ref[0] + t_ref[1]) - x_ref[...]
    h = acc_ref[...] + jnp.dot(tx2, w_ref[2], preferred_element_type=jnp.float32)
    h = jnp.maximum(h + b_ref[...], 0.0)
    out_ref[...] = (
        lax.dot_general(h, wl_ref[...], (((1,), (1,)), ((), ())),
                        preferred_element_type=jnp.float32)
        + bl_ref[...]
    )


def _tc(body, out_shapes):
    return pl.pallas_call(body, out_shape=out_shapes)


def kernel(x, edge_index, W1, b1, W2, b2, gamma, beta, Wlin, blin):
    N, F = x.shape
    E = edge_index.shape[1]
    OUT_F = Wlin.shape[0]
    f32 = jnp.float32

    b1r = b1.reshape(1, -1)
    b2r = b2.reshape(1, -1)
    gr = gamma.reshape(1, -1)
    ber = beta.reshape(1, -1)
    blr = blin.reshape(1, -1)

    src = edge_index[0]
    dst = edge_index[1]

    hist = _hist_call(N, E, F)(src)
    dinv, u0 = _tc(_prep_body, (
        jax.ShapeDtypeStruct((N, _L), f32),
        jax.ShapeDtypeStruct((N, F), f32),
    ))(hist, x)

    scatter = _scatter_call(N, E, F)

    p = scatter(u0, src, dst)
    u1, acc1 = _tc(_mid_body, (
        jax.ShapeDtypeStruct((N, F), f32),
        jax.ShapeDtypeStruct((N, F), f32),
    ))(p, dinv, x, W1)
    q = scatter(u1, src, dst)
    hbn, u0b = _tc(_bn_body, (
        jax.ShapeDtypeStruct((N, F), f32),
        jax.ShapeDtypeStruct((N, F), f32),
    ))(q, dinv, x, acc1, W1, b1r, gr, ber)

    r = scatter(u0b, src, dst)
    u1b, acc2 = _tc(_mid_body, (
        jax.ShapeDtypeStruct((N, F), f32),
        jax.ShapeDtypeStruct((N, F), f32),
    ))(r, dinv, hbn, W2)
    t = scatter(u1b, src, dst)
    out = _tc(_final_body, jax.ShapeDtypeStruct((N, OUT_F), f32))(
        t, dinv, hbn, acc2, W2, b2r, Wlin, blr)
    return out


# speedup vs baseline: 17.8137x; 1.1877x over previous
import functools

import jax
import jax.numpy as jnp
from jax import lax
from jax.experimental import pallas as pl
from jax.experimental.pallas import tpu as pltpu
from jax.experimental.pallas import tpu_sc as plsc

_NC = 2
_NS = 16
_NW = _NC * _NS
_L = 16
_BN_EPS = 1e-5


def _chunk_size(n, nb=1, cap=64):
    for c in range(cap, 7, -8):
        if n % c == 0 and (n // c) % nb == 0:
            return c
    raise ValueError(f"no aligned chunk size for {n}")


def _row_chunk(rows):
    for c in range(64, 7, -8):
        if rows % c == 0:
            return c
    raise ValueError(f"no aligned row chunk for {rows}")


@functools.lru_cache(maxsize=None)
def _hist_call(N, E, HW):
    EPW = E // _NW
    NB = 5
    C = _chunk_size(EPW, nb=NB)
    NCH = EPW // C
    ZR = _row_chunk(N)
    NZ = N // ZR
    NZI = (NZ + _NS - 1) // _NS
    mesh = plsc.VectorSubcoreMesh(core_axis_name="c", subcore_axis_name="s", num_cores=_NC, num_subcores=_NS)

    @functools.partial(
        pl.kernel,
        mesh=mesh,
        out_type=jax.ShapeDtypeStruct((_NC, N, HW), jnp.float32),
        scratch_types=(
            [pltpu.VMEM_SHARED((N, HW), jnp.float32),
             pltpu.VMEM((ZR, HW), jnp.float32),
             pltpu.VMEM((C, HW), jnp.float32)]
            + [pltpu.VMEM((C,), jnp.int32)] * NB
            + [pltpu.SemaphoreType.DMA] * 2
        ),
    )
    def hist(src_hbm, out_hbm, acc_sp, z_v, ones_v, *bufs):
        dstb = bufs[:NB]
        ssem, dsem = bufs[NB:]
        cid = lax.axis_index("c")
        sid = lax.axis_index("s")
        wid = sid * _NC + cid
        base = wid * EPW

        def fill_z(i, _):
            for k in range(HW // _L):
                z_v[i, pl.ds(k * _L, _L)] = jnp.zeros((_L,), jnp.float32)
            return 0

        lax.fori_loop(0, ZR, fill_z, 0)

        def fill_o(i, _):
            for k in range(HW // _L):
                ones_v[i, pl.ds(k * _L, _L)] = jnp.ones((_L,), jnp.float32)
            return 0

        lax.fori_loop(0, C, fill_o, 0)

        def zero_acc(i, _):
            k = i * _NS + sid

            @pl.when(k < NZ)
            def _():
                pltpu.sync_copy(z_v, acc_sp.at[pl.ds(k * ZR, ZR)])

            return 0

        lax.fori_loop(0, NZI, zero_acc, 0)
        plsc.subcore_barrier()

        def dst_cp(j, b):
            return pltpu.make_async_copy(
                src_hbm.at[pl.ds(base + j * C, C)], dstb[b], dsem)

        def sca_start(b):
            pltpu.async_copy(ones_v, acc_sp.at[dstb[b]], ssem, add=True)

        def sca_wait(b):
            pltpu.make_async_copy(ones_v, acc_sp.at[dstb[b]], ssem).wait()

        for m in range(3):
            dst_cp(m, m).start()

        def block(k, _):
            for b in range(NB):
                j = k * NB + b

                @pl.when(j >= 2)
                def _(b2=(b - 2) % NB):
                    sca_wait(b2)

                dst_cp(j, b).wait()
                sca_start(b)

                @pl.when(j + 3 < NCH)
                def _(b2=(b + 3) % NB, j2=j + 3):
                    dst_cp(j2, b2).start()

            return 0

        lax.fori_loop(0, NCH // NB, block, 0)
        sca_wait((NCH - 2) % NB)
        sca_wait((NCH - 1) % NB)
        plsc.subcore_barrier()

        def readout(i, _):
            k = i * _NS + sid

            @pl.when(k < NZ)
            def _():
                pltpu.sync_copy(acc_sp.at[pl.ds(k * ZR, ZR)], z_v)
                pltpu.sync_copy(z_v, out_hbm.at[cid, pl.ds(k * ZR, ZR)])

            return 0

        lax.fori_loop(0, NZI, readout, 0)

    return hist


@functools.lru_cache(maxsize=None)
def _scatter_call(N, E, F):
    EPW = E // _NW
    NB = 5
    C = _chunk_size(EPW, nb=NB)
    NCH = EPW // C
    ZR = _row_chunk(N)
    NZ = N // ZR
    NZI = (NZ + _NS - 1) // _NS
    mesh = plsc.VectorSubcoreMesh(core_axis_name="c", subcore_axis_name="s", num_cores=_NC, num_subcores=_NS)

    @functools.partial(
        pl.kernel,
        mesh=mesh,
        out_type=jax.ShapeDtypeStruct((_NC, N, F), jnp.float32),
        scratch_types=(
            [pltpu.VMEM_SHARED((N, F), jnp.float32),
             pltpu.VMEM((ZR, F), jnp.float32)]
            + [pltpu.VMEM((C,), jnp.int32)] * NB
            + [pltpu.VMEM((C,), jnp.int32)] * NB
            + [pltpu.VMEM((C, F), jnp.float32)] * NB
            + [pltpu.SemaphoreType.DMA] * 4
        ),
    )
    def scat(u_hbm, src_hbm, dst_hbm, out_hbm, acc_sp, z_v, *bufs):
        srcb = bufs[:NB]
        dstb = bufs[NB:2 * NB]
        rows = bufs[2 * NB:3 * NB]
        gsem, ssem, xsem, dsem = bufs[3 * NB:]
        cid = lax.axis_index("c")
        sid = lax.axis_index("s")
        wid = sid * _NC + cid
        base = wid * EPW

        def fill_z(i, _):
            for k in range(F // _L):
                z_v[i, pl.ds(k * _L, _L)] = jnp.zeros((_L,), jnp.float32)
            return 0

        lax.fori_loop(0, ZR, fill_z, 0)

        def zero_acc(i, _):
            k = i * _NS + sid

            @pl.when(k < NZ)
            def _():
                pltpu.sync_copy(z_v, acc_sp.at[pl.ds(k * ZR, ZR)])

            return 0

        lax.fori_loop(0, NZI, zero_acc, 0)
        plsc.subcore_barrier()

        def src_cp(j, b):
            return pltpu.make_async_copy(
                src_hbm.at[pl.ds(base + j * C, C)], srcb[b], xsem)

        def dst_cp(j, b):
            return pltpu.make_async_copy(
                dst_hbm.at[pl.ds(base + j * C, C)], dstb[b], dsem)

        def gat_cp(b):
            return pltpu.make_async_copy(u_hbm.at[srcb[b]], rows[b], gsem)

        def sca_start(b):
            pltpu.async_copy(rows[b], acc_sp.at[dstb[b]], ssem, add=True)

        def sca_wait(b):
            pltpu.make_async_copy(rows[b], acc_sp.at[dstb[b]], ssem).wait()

        for m in range(NB):
            src_cp(m, m).start()
        for m in range(3):
            dst_cp(m, m).start()
        for m in range(3):
            src_cp(m, m).wait()
            gat_cp(m).start()

        def block(k, _):
            for b in range(NB):
                j = k * NB + b

                @pl.when(j >= 2)
                def _(b2=(b - 2) % NB):
                    sca_wait(b2)

                @pl.when(j + 3 < NCH)
                def _(b2=(b + 3) % NB, j2=j + 3):
                    src_cp(j2, b2).wait()
                    gat_cp(b2).start()

                gat_cp(b).wait()
                dst_cp(j, b).wait()
                sca_start(b)

                @pl.when(j + NB < NCH)
                def _(j2=j + NB):
                    src_cp(j2, b).start()

                @pl.when(j + 3 < NCH)
                def _(b2=(b + 3) % NB, j2=j + 3):
                    dst_cp(j2, b2).start()

            return 0

        lax.fori_loop(0, NCH // NB, block, 0)

        sca_wait((NCH - 2) % NB)
        sca_wait((NCH - 1) % NB)
        plsc.subcore_barrier()

        def readout(i, _):
            k = i * _NS + sid

            @pl.when(k < NZ)
            def _():
                pltpu.sync_copy(acc_sp.at[pl.ds(k * ZR, ZR)], z_v)
                pltpu.sync_copy(z_v, out_hbm.at[cid, pl.ds(k * ZR, ZR)])

            return 0

        lax.fori_loop(0, NZI, readout, 0)

    return scat


def _prep_body(hist_ref, x_ref, dinv_ref, u0_ref):
    deg = hist_ref[0, :, :_L] + hist_ref[1, :, :_L]
    dinv = jnp.where(deg > 0, 1.0 / jnp.sqrt(jnp.maximum(deg, 1e-12)), 0.0)
    dinv_ref[...] = dinv
    u0_ref[...] = x_ref[...] * dinv[:, :1]


def _mid_body(p_ref, dinv_ref, x_ref, w_ref, u1_ref, acc_ref):
    d = dinv_ref[:, :1]
    tx1 = -d * (p_ref[0] + p_ref[1])
    u1_ref[...] = d * tx1
    acc_ref[...] = (
        jnp.dot(x_ref[...], w_ref[0], preferred_element_type=jnp.float32)
        + jnp.dot(tx1, w_ref[1], preferred_element_type=jnp.float32)
    )


def _bn_body(q_ref, dinv_ref, x_ref, acc_ref, w_ref, b_ref, g_ref, be_ref,
             hbn_ref, u0_ref):
    d = dinv_ref[:, :1]
    tx2 = -2.0 * d * (q_ref[0] + q_ref[1]) - x_ref[...]
    h = acc_ref[...] + jnp.dot(tx2, w_ref[2], preferred_element_type=jnp.float32)
    h = jnp.maximum(h + b_ref[...], 0.0)
    mean = jnp.mean(h, axis=0, keepdims=True)
    var = jnp.mean((h - mean) ** 2, axis=0, keepdims=True)
    hbn = (h - mean) / jnp.sqrt(var + _BN_EPS) * g_ref[...] + be_ref[...]
    hbn_ref[...] = hbn
    u0_ref[...] = d * hbn


def _final_body(t_ref, dinv_ref, x_ref, acc_ref, w_ref, b_ref, wl_ref, bl_ref,
                out_ref):
    d = dinv_ref[:, :1]
    tx2 = -2.0 * d * (t_ref[0] + t_ref[1]) - x_ref[...]
    h = acc_ref[...] + jnp.dot(tx2, w_ref[2], preferred_element_type=jnp.float32)
    h = jnp.maximum(h + b_ref[...], 0.0)
    out_ref[...] = (
        lax.dot_general(h, wl_ref[...], (((1,), (1,)), ((), ())),
                        preferred_element_type=jnp.float32)
        + bl_ref[...]
    )


def _tc(body, out_shapes):
    return pl.pallas_call(body, out_shape=out_shapes)


def kernel(x, edge_index, W1, b1, W2, b2, gamma, beta, Wlin, blin):
    N, F = x.shape
    E = edge_index.shape[1]
    OUT_F = Wlin.shape[0]
    f32 = jnp.float32

    b1r = b1.reshape(1, -1)
    b2r = b2.reshape(1, -1)
    gr = gamma.reshape(1, -1)
    ber = beta.reshape(1, -1)
    blr = blin.reshape(1, -1)

    src = edge_index[0]
    dst = edge_index[1]

    hist = _hist_call(N, E, F)(src)
    dinv, u0 = _tc(_prep_body, (
        jax.ShapeDtypeStruct((N, _L), f32),
        jax.ShapeDtypeStruct((N, F), f32),
    ))(hist, x)

    scatter = _scatter_call(N, E, F)

    p = scatter(u0, src, dst)
    u1, acc1 = _tc(_mid_body, (
        jax.ShapeDtypeStruct((N, F), f32),
        jax.ShapeDtypeStruct((N, F), f32),
    ))(p, dinv, x, W1)
    q = scatter(u1, src, dst)
    hbn, u0b = _tc(_bn_body, (
        jax.ShapeDtypeStruct((N, F), f32),
        jax.ShapeDtypeStruct((N, F), f32),
    ))(q, dinv, x, acc1, W1, b1r, gr, ber)

    r = scatter(u0b, src, dst)
    u1b, acc2 = _tc(_mid_body, (
        jax.ShapeDtypeStruct((N, F), f32),
        jax.ShapeDtypeStruct((N, F), f32),
    ))(r, dinv, hbn, W2)
    t = scatter(u1b, src, dst)
    out = _tc(_final_body, jax.ShapeDtypeStruct((N, OUT_F), f32))(
        t, dinv, hbn, acc2, W2, b2r, Wlin, blr)
    return out


# speedup vs baseline: 19.6006x; 1.1003x over previous
import functools

import jax
import jax.numpy as jnp
from jax import lax
from jax.experimental import pallas as pl
from jax.experimental.pallas import tpu as pltpu
from jax.experimental.pallas import tpu_sc as plsc

_NC = 2
_NS = 16
_NW = _NC * _NS
_L = 16
_BN_EPS = 1e-5


def _chunk_size(n, nb=1, cap=64):
    for c in range(cap, 7, -8):
        if n % c == 0 and (n // c) % nb == 0:
            return c
    raise ValueError(f"no aligned chunk size for {n}")


def _row_chunk(rows):
    for c in range(64, 7, -8):
        if rows % c == 0:
            return c
    raise ValueError(f"no aligned row chunk for {rows}")


@functools.lru_cache(maxsize=None)
def _hist_call(N, E, HW):
    EPW = E // _NW
    NB = 5
    C = _chunk_size(EPW, nb=NB)
    NCH = EPW // C
    ZR = _row_chunk(N)
    NZ = N // ZR
    NZI = (NZ + _NS - 1) // _NS
    mesh = plsc.VectorSubcoreMesh(core_axis_name="c", subcore_axis_name="s", num_cores=_NC, num_subcores=_NS)

    @functools.partial(
        pl.kernel,
        mesh=mesh,
        out_type=jax.ShapeDtypeStruct((_NC, N, HW), jnp.float32),
        scratch_types=(
            [pltpu.VMEM_SHARED((N, HW), jnp.float32),
             pltpu.VMEM((ZR, HW), jnp.float32),
             pltpu.VMEM((C, HW), jnp.float32)]
            + [pltpu.VMEM((C,), jnp.int32)] * NB
            + [pltpu.SemaphoreType.DMA] * 2
        ),
    )
    def hist(src_hbm, out_hbm, acc_sp, z_v, ones_v, *bufs):
        dstb = bufs[:NB]
        ssem, dsem = bufs[NB:]
        cid = lax.axis_index("c")
        sid = lax.axis_index("s")
        wid = sid * _NC + cid
        base = wid * EPW

        def fill_z(i, _):
            for k in range(HW // _L):
                z_v[i, pl.ds(k * _L, _L)] = jnp.zeros((_L,), jnp.float32)
            return 0

        lax.fori_loop(0, ZR, fill_z, 0)

        def fill_o(i, _):
            for k in range(HW // _L):
                ones_v[i, pl.ds(k * _L, _L)] = jnp.ones((_L,), jnp.float32)
            return 0

        lax.fori_loop(0, C, fill_o, 0)

        def zero_acc(i, _):
            k = i * _NS + sid

            @pl.when(k < NZ)
            def _():
                pltpu.async_copy(z_v, acc_sp.at[pl.ds(k * ZR, ZR)], ssem)

            return 0

        lax.fori_loop(0, NZI, zero_acc, 0)

        def zero_wait(i, _):
            k = i * _NS + sid

            @pl.when(k < NZ)
            def _():
                pltpu.make_async_copy(
                    z_v, acc_sp.at[pl.ds(k * ZR, ZR)], ssem).wait()

            return 0

        lax.fori_loop(0, NZI, zero_wait, 0)
        plsc.subcore_barrier()

        def dst_cp(j, b):
            return pltpu.make_async_copy(
                src_hbm.at[pl.ds(base + j * C, C)], dstb[b], dsem)

        def sca_start(b):
            pltpu.async_copy(ones_v, acc_sp.at[dstb[b]], ssem, add=True)

        def sca_wait(b):
            pltpu.make_async_copy(ones_v, acc_sp.at[dstb[b]], ssem).wait()

        for m in range(3):
            dst_cp(m, m).start()

        def block(k, _):
            for b in range(NB):
                j = k * NB + b

                @pl.when(j >= 2)
                def _(b2=(b - 2) % NB):
                    sca_wait(b2)

                dst_cp(j, b).wait()
                sca_start(b)

                @pl.when(j + 3 < NCH)
                def _(b2=(b + 3) % NB, j2=j + 3):
                    dst_cp(j2, b2).start()

            return 0

        lax.fori_loop(0, NCH // NB, block, 0)
        sca_wait((NCH - 2) % NB)
        sca_wait((NCH - 1) % NB)
        plsc.subcore_barrier()

        def readout(i, _):
            k = i * _NS + sid

            @pl.when(k < NZ)
            def _():
                pltpu.async_copy(acc_sp.at[pl.ds(k * ZR, ZR)],
                                 out_hbm.at[cid, pl.ds(k * ZR, ZR)], dsem)

            return 0

        lax.fori_loop(0, NZI, readout, 0)

        def read_wait(i, _):
            k = i * _NS + sid

            @pl.when(k < NZ)
            def _():
                pltpu.make_async_copy(
                    acc_sp.at[pl.ds(k * ZR, ZR)],
                    out_hbm.at[cid, pl.ds(k * ZR, ZR)], dsem).wait()

            return 0

        lax.fori_loop(0, NZI, read_wait, 0)

    return hist


@functools.lru_cache(maxsize=None)
def _scatter_call(N, E, F):
    EPW = E // _NW
    NB = 5
    C = _chunk_size(EPW, nb=NB)
    NCH = EPW // C
    ZR = _row_chunk(N)
    NZ = N // ZR
    NZI = (NZ + _NS - 1) // _NS
    mesh = plsc.VectorSubcoreMesh(core_axis_name="c", subcore_axis_name="s", num_cores=_NC, num_subcores=_NS)

    @functools.partial(
        pl.kernel,
        mesh=mesh,
        out_type=jax.ShapeDtypeStruct((_NC, N, F), jnp.float32),
        scratch_types=(
            [pltpu.VMEM_SHARED((N, F), jnp.float32),
             pltpu.VMEM((ZR, F), jnp.float32)]
            + [pltpu.VMEM((C,), jnp.int32)] * NB
            + [pltpu.VMEM((C,), jnp.int32)] * NB
            + [pltpu.VMEM((C, F), jnp.float32)] * NB
            + [pltpu.SemaphoreType.DMA] * 4
        ),
    )
    def scat(u_hbm, src_hbm, dst_hbm, out_hbm, acc_sp, z_v, *bufs):
        srcb = bufs[:NB]
        dstb = bufs[NB:2 * NB]
        rows = bufs[2 * NB:3 * NB]
        gsem, ssem, xsem, dsem = bufs[3 * NB:]
        cid = lax.axis_index("c")
        sid = lax.axis_index("s")
        wid = sid * _NC + cid
        base = wid * EPW

        def fill_z(i, _):
            for k in range(F // _L):
                z_v[i, pl.ds(k * _L, _L)] = jnp.zeros((_L,), jnp.float32)
            return 0

        lax.fori_loop(0, ZR, fill_z, 0)

        def zero_acc(i, _):
            k = i * _NS + sid

            @pl.when(k < NZ)
            def _():
                pltpu.async_copy(z_v, acc_sp.at[pl.ds(k * ZR, ZR)], ssem)

            return 0

        lax.fori_loop(0, NZI, zero_acc, 0)

        def zero_wait(i, _):
            k = i * _NS + sid

            @pl.when(k < NZ)
            def _():
                pltpu.make_async_copy(
                    z_v, acc_sp.at[pl.ds(k * ZR, ZR)], ssem).wait()

            return 0

        lax.fori_loop(0, NZI, zero_wait, 0)
        plsc.subcore_barrier()

        def src_cp(j, b):
            return pltpu.make_async_copy(
                src_hbm.at[pl.ds(base + j * C, C)], srcb[b], xsem)

        def dst_cp(j, b):
            return pltpu.make_async_copy(
                dst_hbm.at[pl.ds(base + j * C, C)], dstb[b], dsem)

        def gat_cp(b):
            return pltpu.make_async_copy(u_hbm.at[srcb[b]], rows[b], gsem)

        def sca_start(b):
            pltpu.async_copy(rows[b], acc_sp.at[dstb[b]], ssem, add=True)

        def sca_wait(b):
            pltpu.make_async_copy(rows[b], acc_sp.at[dstb[b]], ssem).wait()

        for m in range(NB):
            src_cp(m, m).start()
        for m in range(3):
            dst_cp(m, m).start()
        for m in range(3):
            src_cp(m, m).wait()
            gat_cp(m).start()

        def block(k, _):
            for b in range(NB):
                j = k * NB + b

                @pl.when(j >= 2)
                def _(b2=(b - 2) % NB):
                    sca_wait(b2)

                @pl.when(j + 3 < NCH)
                def _(b2=(b + 3) % NB, j2=j + 3):
                    src_cp(j2, b2).wait()
                    gat_cp(b2).start()

                gat_cp(b).wait()
                dst_cp(j, b).wait()
                sca_start(b)

                @pl.when(j + NB < NCH)
                def _(j2=j + NB):
                    src_cp(j2, b).start()

                @pl.when(j + 3 < NCH)
                def _(b2=(b + 3) % NB, j2=j + 3):
                    dst_cp(j2, b2).start()

            return 0

        lax.fori_loop(0, NCH // NB, block, 0)

        sca_wait((NCH - 2) % NB)
        sca_wait((NCH - 1) % NB)
        plsc.subcore_barrier()

        def readout(i, _):
            k = i * _NS + sid

            @pl.when(k < NZ)
            def _():
                pltpu.async_copy(acc_sp.at[pl.ds(k * ZR, ZR)],
                                 out_hbm.at[cid, pl.ds(k * ZR, ZR)], dsem)

            return 0

        lax.fori_loop(0, NZI, readout, 0)

        def read_wait(i, _):
            k = i * _NS + sid

            @pl.when(k < NZ)
            def _():
                pltpu.make_async_copy(
                    acc_sp.at[pl.ds(k * ZR, ZR)],
                    out_hbm.at[cid, pl.ds(k * ZR, ZR)], dsem).wait()

            return 0

        lax.fori_loop(0, NZI, read_wait, 0)

    return scat


def _prep_body(hist_ref, x_ref, dinv_ref, u0_ref):
    deg = hist_ref[0, :, :_L] + hist_ref[1, :, :_L]
    dinv = jnp.where(deg > 0, 1.0 / jnp.sqrt(jnp.maximum(deg, 1e-12)), 0.0)
    dinv_ref[...] = dinv
    u0_ref[...] = x_ref[...] * dinv[:, :1]


def _mid_body(p_ref, dinv_ref, x_ref, w_ref, u1_ref, acc_ref):
    d = dinv_ref[:, :1]
    tx1 = -d * (p_ref[0] + p_ref[1])
    u1_ref[...] = d * tx1
    acc_ref[...] = (
        jnp.dot(x_ref[...], w_ref[0], preferred_element_type=jnp.float32)
        + jnp.dot(tx1, w_ref[1], preferred_element_type=jnp.float32)
    )


def _bn_body(q_ref, dinv_ref, x_ref, acc_ref, w_ref, b_ref, g_ref, be_ref,
             hbn_ref, u0_ref):
    d = dinv_ref[:, :1]
    tx2 = -2.0 * d * (q_ref[0] + q_ref[1]) - x_ref[...]
    h = acc_ref[...] + jnp.dot(tx2, w_ref[2], preferred_element_type=jnp.float32)
    h = jnp.maximum(h + b_ref[...], 0.0)
    mean = jnp.mean(h, axis=0, keepdims=True)
    var = jnp.mean((h - mean) ** 2, axis=0, keepdims=True)
    hbn = (h - mean) / jnp.sqrt(var + _BN_EPS) * g_ref[...] + be_ref[...]
    hbn_ref[...] = hbn
    u0_ref[...] = d * hbn


def _final_body(t_ref, dinv_ref, x_ref, acc_ref, w_ref, b_ref, wl_ref, bl_ref,
                out_ref):
    d = dinv_ref[:, :1]
    tx2 = -2.0 * d * (t_ref[0] + t_ref[1]) - x_ref[...]
    h = acc_ref[...] + jnp.dot(tx2, w_ref[2], preferred_element_type=jnp.float32)
    h = jnp.maximum(h + b_ref[...], 0.0)
    out_ref[...] = (
        lax.dot_general(h, wl_ref[...], (((1,), (1,)), ((), ())),
                        preferred_element_type=jnp.float32)
        + bl_ref[...]
    )


def _tc(body, out_shapes):
    return pl.pallas_call(body, out_shape=out_shapes)


def kernel(x, edge_index, W1, b1, W2, b2, gamma, beta, Wlin, blin):
    N, F = x.shape
    E = edge_index.shape[1]
    OUT_F = Wlin.shape[0]
    f32 = jnp.float32

    b1r = b1.reshape(1, -1)
    b2r = b2.reshape(1, -1)
    gr = gamma.reshape(1, -1)
    ber = beta.reshape(1, -1)
    blr = blin.reshape(1, -1)

    src = edge_index[0]
    dst = edge_index[1]

    hist = _hist_call(N, E, F)(src)
    dinv, u0 = _tc(_prep_body, (
        jax.ShapeDtypeStruct((N, _L), f32),
        jax.ShapeDtypeStruct((N, F), f32),
    ))(hist, x)

    scatter = _scatter_call(N, E, F)

    p = scatter(u0, src, dst)
    u1, acc1 = _tc(_mid_body, (
        jax.ShapeDtypeStruct((N, F), f32),
        jax.ShapeDtypeStruct((N, F), f32),
    ))(p, dinv, x, W1)
    q = scatter(u1, src, dst)
    hbn, u0b = _tc(_bn_body, (
        jax.ShapeDtypeStruct((N, F), f32),
        jax.ShapeDtypeStruct((N, F), f32),
    ))(q, dinv, x, acc1, W1, b1r, gr, ber)

    r = scatter(u0b, src, dst)
    u1b, acc2 = _tc(_mid_body, (
        jax.ShapeDtypeStruct((N, F), f32),
        jax.ShapeDtypeStruct((N, F), f32),
    ))(r, dinv, hbn, W2)
    t = scatter(u1b, src, dst)
    out = _tc(_final_body, jax.ShapeDtypeStruct((N, OUT_F), f32))(
        t, dinv, hbn, acc2, W2, b2r, Wlin, blr)
    return out


# speedup vs baseline: 20.3187x; 1.0366x over previous
import functools

import jax
import jax.numpy as jnp
from jax import lax
from jax.experimental import pallas as pl
from jax.experimental.pallas import tpu as pltpu
from jax.experimental.pallas import tpu_sc as plsc

_NC = 2
_NS = 16
_NW = _NC * _NS
_L = 16
_BN_EPS = 1e-5


def _chunk_size(n, nb=1, cap=64):
    for c in range(cap, 7, -8):
        if n % c == 0 and (n // c) % nb == 0:
            return c
    raise ValueError(f"no aligned chunk size for {n}")


def _row_chunk(rows):
    for c in range(64, 7, -8):
        if rows % c == 0:
            return c
    raise ValueError(f"no aligned row chunk for {rows}")


@functools.lru_cache(maxsize=None)
def _hist_call(N, E, HW):
    EPW = E // _NW
    NB = 5
    C = _chunk_size(EPW, nb=NB)
    NCH = EPW // C
    ZR = _row_chunk(N)
    NZ = N // ZR
    NZI = (NZ + _NS - 1) // _NS
    mesh = plsc.VectorSubcoreMesh(core_axis_name="c", subcore_axis_name="s", num_cores=_NC, num_subcores=_NS)

    @functools.partial(
        pl.kernel,
        mesh=mesh,
        out_type=jax.ShapeDtypeStruct((_NC, N, HW), jnp.float32),
        scratch_types=(
            [pltpu.VMEM_SHARED((N, HW), jnp.float32),
             pltpu.VMEM((ZR, HW), jnp.float32),
             pltpu.VMEM((C, HW), jnp.float32)]
            + [pltpu.VMEM((C,), jnp.int32)] * NB
            + [pltpu.SemaphoreType.DMA] * 2
        ),
    )
    def hist(src_hbm, out_hbm, acc_sp, z_v, ones_v, *bufs):
        dstb = bufs[:NB]
        ssem, dsem = bufs[NB:]
        cid = lax.axis_index("c")
        sid = lax.axis_index("s")
        wid = sid * _NC + cid
        base = wid * EPW

        def fill_z(i, _):
            for k in range(HW // _L):
                z_v[i, pl.ds(k * _L, _L)] = jnp.zeros((_L,), jnp.float32)
            return 0

        lax.fori_loop(0, ZR, fill_z, 0)

        def fill_o(i, _):
            for k in range(HW // _L):
                ones_v[i, pl.ds(k * _L, _L)] = jnp.ones((_L,), jnp.float32)
            return 0

        lax.fori_loop(0, C, fill_o, 0)

        def zero_acc(i, _):
            k = i * _NS + sid

            @pl.when(k < NZ)
            def _():
                pltpu.async_copy(z_v, acc_sp.at[pl.ds(k * ZR, ZR)], ssem)

            return 0

        lax.fori_loop(0, NZI, zero_acc, 0)

        def zero_wait(i, _):
            k = i * _NS + sid

            @pl.when(k < NZ)
            def _():
                pltpu.make_async_copy(
                    z_v, acc_sp.at[pl.ds(k * ZR, ZR)], ssem).wait()

            return 0

        lax.fori_loop(0, NZI, zero_wait, 0)
        plsc.subcore_barrier()

        def dst_cp(j, b):
            return pltpu.make_async_copy(
                src_hbm.at[pl.ds(base + j * C, C)], dstb[b], dsem)

        def sca_start(b):
            pltpu.async_copy(ones_v, acc_sp.at[dstb[b]], ssem, add=True)

        def sca_wait(b):
            pltpu.make_async_copy(ones_v, acc_sp.at[dstb[b]], ssem).wait()

        for m in range(3):
            dst_cp(m, m).start()

        def block(k, _):
            for b in range(NB):
                j = k * NB + b

                @pl.when(j >= 2)
                def _(b2=(b - 2) % NB):
                    sca_wait(b2)

                dst_cp(j, b).wait()
                sca_start(b)

                @pl.when(j + 3 < NCH)
                def _(b2=(b + 3) % NB, j2=j + 3):
                    dst_cp(j2, b2).start()

            return 0

        lax.fori_loop(0, NCH // NB, block, 0)
        sca_wait((NCH - 2) % NB)
        sca_wait((NCH - 1) % NB)
        plsc.subcore_barrier()

        def readout(i, _):
            k = i * _NS + sid

            @pl.when(k < NZ)
            def _():
                pltpu.async_copy(acc_sp.at[pl.ds(k * ZR, ZR)],
                                 out_hbm.at[cid, pl.ds(k * ZR, ZR)], dsem)

            return 0

        lax.fori_loop(0, NZI, readout, 0)

        def read_wait(i, _):
            k = i * _NS + sid

            @pl.when(k < NZ)
            def _():
                pltpu.make_async_copy(
                    acc_sp.at[pl.ds(k * ZR, ZR)],
                    out_hbm.at[cid, pl.ds(k * ZR, ZR)], dsem).wait()

            return 0

        lax.fori_loop(0, NZI, read_wait, 0)

    return hist


@functools.lru_cache(maxsize=None)
def _scatter_call(N, E, F):
    EPW = E // _NW
    NB = 5
    C = _chunk_size(EPW, nb=NB)
    NCH = EPW // C
    ZR = _row_chunk(N)
    NZ = N // ZR
    NZI = (NZ + _NS - 1) // _NS
    mesh = plsc.VectorSubcoreMesh(core_axis_name="c", subcore_axis_name="s", num_cores=_NC, num_subcores=_NS)

    @functools.partial(
        pl.kernel,
        mesh=mesh,
        out_type=jax.ShapeDtypeStruct((_NC, N, F), jnp.float32),
        scratch_types=(
            [pltpu.VMEM_SHARED((N, F), jnp.float32),
             pltpu.VMEM((EPW,), jnp.int32)]
            + [pltpu.VMEM((C,), jnp.int32)] * NB
            + [pltpu.VMEM((C, F), jnp.float32)] * NB
            + [pltpu.SemaphoreType.DMA] * 4
        ),
    )
    def scat(u_hbm, src_hbm, dst_hbm, out_hbm, acc_sp, srcf, *bufs):
        dstb = bufs[:NB]
        rows = bufs[NB:2 * NB]
        gsem, ssem, xsem, dsem = bufs[2 * NB:]
        cid = lax.axis_index("c")
        sid = lax.axis_index("s")
        wid = sid * _NC + cid
        base = wid * EPW

        pltpu.async_copy(src_hbm.at[pl.ds(base, EPW)], srcf, xsem)

        assert ZR <= C
        z_v = rows[0].at[pl.ds(0, ZR)]
        ZRF = C

        def fill_z(i, _):
            for k in range(F // _L):
                rows[0][i, pl.ds(k * _L, _L)] = jnp.zeros((_L,), jnp.float32)
            return 0

        lax.fori_loop(0, ZRF, fill_z, 0)

        def zero_acc(i, _):
            k = i * _NS + sid

            @pl.when(k < NZ)
            def _():
                pltpu.async_copy(z_v, acc_sp.at[pl.ds(k * ZR, ZR)], ssem)

            return 0

        lax.fori_loop(0, NZI, zero_acc, 0)

        def zero_wait(i, _):
            k = i * _NS + sid

            @pl.when(k < NZ)
            def _():
                pltpu.make_async_copy(
                    z_v, acc_sp.at[pl.ds(k * ZR, ZR)], ssem).wait()

            return 0

        lax.fori_loop(0, NZI, zero_wait, 0)
        plsc.subcore_barrier()

        def dst_cp(j, b):
            return pltpu.make_async_copy(
                dst_hbm.at[pl.ds(base + j * C, C)], dstb[b], dsem)

        def gat_cp(j, b):
            return pltpu.make_async_copy(
                u_hbm.at[srcf.at[pl.ds(j * C, C)]], rows[b], gsem)

        def sca_start(b):
            pltpu.async_copy(rows[b], acc_sp.at[dstb[b]], ssem, add=True)

        def sca_wait(b):
            pltpu.make_async_copy(rows[b], acc_sp.at[dstb[b]], ssem).wait()

        for m in range(3):
            dst_cp(m, m).start()
        pltpu.make_async_copy(
            src_hbm.at[pl.ds(base, EPW)], srcf, xsem).wait()
        for m in range(3):
            gat_cp(m, m).start()

        def block(k, _):
            for b in range(NB):
                j = k * NB + b

                @pl.when(j >= 2)
                def _(b2=(b - 2) % NB):
                    sca_wait(b2)

                @pl.when(j + 3 < NCH)
                def _(b2=(b + 3) % NB, j2=j + 3):
                    gat_cp(j2, b2).start()

                gat_cp(j, b).wait()
                dst_cp(j, b).wait()
                sca_start(b)

                @pl.when(j + 3 < NCH)
                def _(b2=(b + 3) % NB, j2=j + 3):
                    dst_cp(j2, b2).start()

            return 0

        lax.fori_loop(0, NCH // NB, block, 0)

        sca_wait((NCH - 2) % NB)
        sca_wait((NCH - 1) % NB)
        plsc.subcore_barrier()

        def readout(i, _):
            k = i * _NS + sid

            @pl.when(k < NZ)
            def _():
                pltpu.async_copy(acc_sp.at[pl.ds(k * ZR, ZR)],
                                 out_hbm.at[cid, pl.ds(k * ZR, ZR)], dsem)

            return 0

        lax.fori_loop(0, NZI, readout, 0)

        def read_wait(i, _):
            k = i * _NS + sid

            @pl.when(k < NZ)
            def _():
                pltpu.make_async_copy(
                    acc_sp.at[pl.ds(k * ZR, ZR)],
                    out_hbm.at[cid, pl.ds(k * ZR, ZR)], dsem).wait()

            return 0

        lax.fori_loop(0, NZI, read_wait, 0)

    return scat


def _prep_body(hist_ref, x_ref, dinv_ref, u0_ref):
    deg = hist_ref[0, :, :_L] + hist_ref[1, :, :_L]
    dinv = jnp.where(deg > 0, 1.0 / jnp.sqrt(jnp.maximum(deg, 1e-12)), 0.0)
    dinv_ref[...] = dinv
    u0_ref[...] = x_ref[...] * dinv[:, :1]


def _mid_body(p_ref, dinv_ref, x_ref, w_ref, u1_ref, acc_ref):
    d = dinv_ref[:, :1]
    tx1 = -d * (p_ref[0] + p_ref[1])
    u1_ref[...] = d * tx1
    acc_ref[...] = (
        jnp.dot(x_ref[...], w_ref[0], preferred_element_type=jnp.float32)
        + jnp.dot(tx1, w_ref[1], preferred_element_type=jnp.float32)
    )


def _bn_body(q_ref, dinv_ref, x_ref, acc_ref, w_ref, b_ref, g_ref, be_ref,
             hbn_ref, u0_ref):
    d = dinv_ref[:, :1]
    tx2 = -2.0 * d * (q_ref[0] + q_ref[1]) - x_ref[...]
    h = acc_ref[...] + jnp.dot(tx2, w_ref[2], preferred_element_type=jnp.float32)
    h = jnp.maximum(h + b_ref[...], 0.0)
    mean = jnp.mean(h, axis=0, keepdims=True)
    var = jnp.mean((h - mean) ** 2, axis=0, keepdims=True)
    hbn = (h - mean) / jnp.sqrt(var + _BN_EPS) * g_ref[...] + be_ref[...]
    hbn_ref[...] = hbn
    u0_ref[...] = d * hbn


def _final_body(t_ref, dinv_ref, x_ref, acc_ref, w_ref, b_ref, wl_ref, bl_ref,
                out_ref):
    d = dinv_ref[:, :1]
    tx2 = -2.0 * d * (t_ref[0] + t_ref[1]) - x_ref[...]
    h = acc_ref[...] + jnp.dot(tx2, w_ref[2], preferred_element_type=jnp.float32)
    h = jnp.maximum(h + b_ref[...], 0.0)
    out_ref[...] = (
        lax.dot_general(h, wl_ref[...], (((1,), (1,)), ((), ())),
                        preferred_element_type=jnp.float32)
        + bl_ref[...]
    )


def _tc(body, out_shapes):
    return pl.pallas_call(body, out_shape=out_shapes)


def kernel(x, edge_index, W1, b1, W2, b2, gamma, beta, Wlin, blin):
    N, F = x.shape
    E = edge_index.shape[1]
    OUT_F = Wlin.shape[0]
    f32 = jnp.float32

    b1r = b1.reshape(1, -1)
    b2r = b2.reshape(1, -1)
    gr = gamma.reshape(1, -1)
    ber = beta.reshape(1, -1)
    blr = blin.reshape(1, -1)

    src = edge_index[0]
    dst = edge_index[1]

    hist = _hist_call(N, E, F)(src)
    dinv, u0 = _tc(_prep_body, (
        jax.ShapeDtypeStruct((N, _L), f32),
        jax.ShapeDtypeStruct((N, F), f32),
    ))(hist, x)

    scatter = _scatter_call(N, E, F)

    p = scatter(u0, src, dst)
    u1, acc1 = _tc(_mid_body, (
        jax.ShapeDtypeStruct((N, F), f32),
        jax.ShapeDtypeStruct((N, F), f32),
    ))(p, dinv, x, W1)
    q = scatter(u1, src, dst)
    hbn, u0b = _tc(_bn_body, (
        jax.ShapeDtypeStruct((N, F), f32),
        jax.ShapeDtypeStruct((N, F), f32),
    ))(q, dinv, x, acc1, W1, b1r, gr, ber)

    r = scatter(u0b, src, dst)
    u1b, acc2 = _tc(_mid_body, (
        jax.ShapeDtypeStruct((N, F), f32),
        jax.ShapeDtypeStruct((N, F), f32),
    ))(r, dinv, hbn, W2)
    t = scatter(u1b, src, dst)
    out = _tc(_final_body, jax.ShapeDtypeStruct((N, OUT_F), f32))(
        t, dinv, hbn, acc2, W2, b2r, Wlin, blr)
    return out
